# Initial kernel scaffold; baseline (speedup 1.0000x reference)
import jax
import jax.numpy as jnp
from jax.experimental import pallas as pl


def kernel(x, edge_index, features, batch, W1_rel, b1_rel, W1_root, n1_w, n1_b, m1_W0, m1_b0, m1_ln0_w, m1_ln0_b, m1_W1, m1_b1, W2_rel, b2_rel, W2_root, n2_w, n2_b, m2_W0, m2_b0, m2_ln0_w, m2_ln0_b, m2_W1, m2_b1, m2_ln1_w, m2_ln1_b, m2_W2, m2_b2, m2_ln2_w, m2_ln2_b, m2_W3, m2_b3):
    raise NotImplementedError("write your pallas kernel here")


# speedup vs baseline: 4.6751x; 4.6751x over previous
import functools

import jax
import jax.numpy as jnp
from jax import lax
from jax.experimental import pallas as pl
from jax.experimental.pallas import tpu as pltpu
from jax.experimental.pallas import tpu_sc as plsc

N = 10000
E = 320000
D_IN = 128
H = 128
F_G = 16
B = 64
OUT = 2
EPS = 1e-5

NC = 2
NS = 16
CHUNK = 80
EDGES_PER_TILE = E // (NC * NS)
CHUNKS_PER_TILE = EDGES_PER_TILE // CHUNK
ROWS_PER_TILE = 624
TAIL_ROWS = N - NS * ROWS_PER_TILE
ZROWS = 208


def _segment_sum_sc(x, src, dst, zeros):
    mesh = plsc.VectorSubcoreMesh(core_axis_name="c", subcore_axis_name="s")

    @functools.partial(
        pl.kernel,
        mesh=mesh,
        out_type=jax.ShapeDtypeStruct((NC, N, H), jnp.float32),
        scratch_types=[
            pltpu.VMEM_SHARED((N, H), jnp.float32),
            pltpu.VMEM((CHUNK,), jnp.int32),
            pltpu.VMEM((CHUNK,), jnp.int32),
            pltpu.VMEM((CHUNK, H), jnp.float32),
            pltpu.SemaphoreType.DMA,
        ],
    )
    def seg_kernel(x_hbm, src_hbm, dst_hbm, z_hbm, out_hbm,
                   acc, src_v, dst_v, rows_v, sem):
        c = lax.axis_index("c")
        s = lax.axis_index("s")
        wid = c * NS + s
        r0 = s * ROWS_PER_TILE
        for z in range(ROWS_PER_TILE // ZROWS):
            pltpu.sync_copy(z_hbm, acc.at[pl.ds(r0 + z * ZROWS, ZROWS)])

        @pl.when(s == NS - 1)
        def _zero_tail():
            pltpu.sync_copy(z_hbm.at[pl.ds(0, TAIL_ROWS)],
                            acc.at[pl.ds(NS * ROWS_PER_TILE, TAIL_ROWS)])

        plsc.subcore_barrier()

        base = wid * EDGES_PER_TILE

        def body(i, carry):
            off = base + i * CHUNK
            pltpu.sync_copy(src_hbm.at[pl.ds(off, CHUNK)], src_v)
            pltpu.sync_copy(dst_hbm.at[pl.ds(off, CHUNK)], dst_v)
            pltpu.async_copy(x_hbm.at[src_v], rows_v, sem).wait()
            pltpu.sync_copy(rows_v, acc.at[dst_v], add=True)
            return carry

        lax.fori_loop(0, CHUNKS_PER_TILE, body, 0)
        plsc.subcore_barrier()
        pltpu.sync_copy(
            acc.at[pl.ds(r0, ROWS_PER_TILE)],
            out_hbm.at[c, pl.ds(r0, ROWS_PER_TILE)])

        @pl.when(s == NS - 1)
        def _out_tail():
            pltpu.sync_copy(
                acc.at[pl.ds(NS * ROWS_PER_TILE, TAIL_ROWS)],
                out_hbm.at[c, pl.ds(NS * ROWS_PER_TILE, TAIL_ROWS)])

    return seg_kernel(x, src, dst, zeros)


def _dense1_body(acc0, acc1, x, batch, features,
                 w1rel, b1rel, w1root, n1w, n1b,
                 m1w0, m1b0, m1ln0w, m1ln0b, m1w1, m1b1,
                 h_out, g_out):
    agg = acc0[...] + acc1[...]
    t = (jnp.dot(agg, w1rel[...], preferred_element_type=jnp.float32)
         + b1rel[...]
         + jnp.dot(x[...], w1root[...], preferred_element_type=jnp.float32))
    m = jnp.mean(t)
    d = t - m
    v = jnp.mean(d * d)
    h = jnp.maximum(d / jnp.sqrt(v + EPS) * n1w[...] + n1b[...], 0.0)
    h_out[...] = h

    bat = batch[...]
    rows = lax.broadcasted_iota(jnp.int32, (B, N), 0)
    onehot = (rows == bat).astype(jnp.float32)
    psum = jnp.dot(onehot, h, preferred_element_type=jnp.float32)
    cnt = jnp.sum(onehot, axis=1, keepdims=True)
    pooled = psum / jnp.maximum(cnt, 1.0)

    z0 = jnp.concatenate([pooled, features[...]], axis=1)
    z1 = (jnp.dot(z0, m1w0[...], preferred_element_type=jnp.float32)
          + m1b0[...])
    mu = jnp.mean(z1, axis=1, keepdims=True)
    dv = z1 - mu
    var = jnp.mean(dv * dv, axis=1, keepdims=True)
    z1 = jnp.maximum(dv / jnp.sqrt(var + EPS) * m1ln0w[...] + m1ln0b[...], 0.0)
    g_out[...] = (jnp.dot(z1, m1w1[...], preferred_element_type=jnp.float32)
                  + m1b1[...])


def _dense1(acc0, acc1, x, batch, features, w1rel, b1rel, w1root, n1w, n1b,
            m1w0, m1b0, m1ln0w, m1ln0b, m1w1, m1b1):
    return pl.pallas_call(
        _dense1_body,
        out_shape=[
            jax.ShapeDtypeStruct((N, H), jnp.float32),
            jax.ShapeDtypeStruct((B, 2 * H), jnp.float32),
        ],
    )(acc0, acc1, x, batch, features, w1rel, b1rel, w1root, n1w, n1b,
      m1w0, m1b0, m1ln0w, m1ln0b, m1w1, m1b1)


def _dense2_body(acc0, acc1, h, g, batch, features,
                 w2rel, b2rel, w2root, n2w, n2b,
                 m2w0, m2b0, m2ln0w, m2ln0b,
                 m2w1, m2b1, m2ln1w, m2ln1b,
                 m2w2, m2b2, m2ln2w, m2ln2b,
                 m2w3, m2b3, out):
    agg = acc0[...] + acc1[...]
    t = (jnp.dot(agg, w2rel[...], preferred_element_type=jnp.float32)
         + b2rel[...]
         + jnp.dot(h[...], w2root[...], preferred_element_type=jnp.float32))
    m = jnp.mean(t)
    d = t - m
    v = jnp.mean(d * d)
    h2 = jnp.maximum(d / jnp.sqrt(v + EPS) * n2w[...] + n2b[...], 0.0)

    bat = batch[...]
    rows = lax.broadcasted_iota(jnp.int32, (B, N), 0)
    onehot = (rows == bat).astype(jnp.float32)
    psum = jnp.dot(onehot, h2, preferred_element_type=jnp.float32)
    cnt = jnp.sum(onehot, axis=1, keepdims=True)
    pooled = psum / jnp.maximum(cnt, 1.0)

    z = jnp.concatenate([pooled, g[...], features[...]], axis=1)

    def ln_relu(zz, w, b):
        mu = jnp.mean(zz, axis=1, keepdims=True)
        dv = zz - mu
        var = jnp.mean(dv * dv, axis=1, keepdims=True)
        return jnp.maximum(dv / jnp.sqrt(var + EPS) * w + b, 0.0)

    z = ln_relu(jnp.dot(z, m2w0[...], preferred_element_type=jnp.float32)
                + m2b0[...], m2ln0w[...], m2ln0b[...])
    z = ln_relu(jnp.dot(z, m2w1[...], preferred_element_type=jnp.float32)
                + m2b1[...], m2ln1w[...], m2ln1b[...])
    z = ln_relu(jnp.dot(z, m2w2[...], preferred_element_type=jnp.float32)
                + m2b2[...], m2ln2w[...], m2ln2b[...])
    out[...] = (jnp.dot(z, m2w3[...], preferred_element_type=jnp.float32)
                + m2b3[...])


def _dense2(acc0, acc1, h, g, batch, features, w2rel, b2rel, w2root, n2w, n2b,
            m2w0, m2b0, m2ln0w, m2ln0b, m2w1, m2b1, m2ln1w, m2ln1b,
            m2w2, m2b2, m2ln2w, m2ln2b, m2w3, m2b3):
    return pl.pallas_call(
        _dense2_body,
        out_shape=jax.ShapeDtypeStruct((B, OUT), jnp.float32),
    )(acc0, acc1, h, g, batch, features, w2rel, b2rel, w2root, n2w, n2b,
      m2w0, m2b0, m2ln0w, m2ln0b, m2w1, m2b1, m2ln1w, m2ln1b,
      m2w2, m2b2, m2ln2w, m2ln2b, m2w3, m2b3)


def kernel(x, edge_index, features, batch,
           W1_rel, b1_rel, W1_root, n1_w, n1_b,
           m1_W0, m1_b0, m1_ln0_w, m1_ln0_b, m1_W1, m1_b1,
           W2_rel, b2_rel, W2_root, n2_w, n2_b,
           m2_W0, m2_b0, m2_ln0_w, m2_ln0_b, m2_W1, m2_b1,
           m2_ln1_w, m2_ln1_b, m2_W2, m2_b2, m2_ln2_w, m2_ln2_b,
           m2_W3, m2_b3):
    src = edge_index[0]
    dst = edge_index[1]
    zeros = jnp.zeros((ZROWS, H), jnp.float32)
    batch2d = batch.reshape(1, N)

    r1 = lambda a: a.reshape(1, -1)

    agg1 = _segment_sum_sc(x, src, dst, zeros)
    h, g = _dense1(agg1[0], agg1[1], x, batch2d, features,
                   W1_rel, r1(b1_rel), W1_root, r1(n1_w), r1(n1_b),
                   m1_W0, r1(m1_b0), r1(m1_ln0_w), r1(m1_ln0_b),
                   m1_W1, r1(m1_b1))
    agg2 = _segment_sum_sc(h, src, dst, zeros)
    out = _dense2(agg2[0], agg2[1], h, g, batch2d, features,
                  W2_rel, r1(b2_rel), W2_root, r1(n2_w), r1(n2_b),
                  m2_W0, r1(m2_b0), r1(m2_ln0_w), r1(m2_ln0_b),
                  m2_W1, r1(m2_b1), r1(m2_ln1_w), r1(m2_ln1_b),
                  m2_W2, r1(m2_b2), r1(m2_ln2_w), r1(m2_ln2_b),
                  m2_W3, r1(m2_b3))
    return out


# speedup vs baseline: 10.3197x; 2.2074x over previous
import functools

import jax
import jax.numpy as jnp
from jax import lax
from jax.experimental import pallas as pl
from jax.experimental.pallas import tpu as pltpu
from jax.experimental.pallas import tpu_sc as plsc

N = 10000
E = 320000
D_IN = 128
H = 128
F_G = 16
B = 64
OUT = 2
EPS = 1e-5

NC = 2
NS = 16
CHUNK = 80
EDGES_PER_TILE = E // (NC * NS)
CHUNKS_PER_TILE = EDGES_PER_TILE // CHUNK
ROWS_PER_TILE = 624
TAIL_ROWS = N - NS * ROWS_PER_TILE
ZROWS = 208


def _segment_sum_sc(x, src, dst, zeros):
    mesh = plsc.VectorSubcoreMesh(core_axis_name="c", subcore_axis_name="s")

    @functools.partial(
        pl.kernel,
        mesh=mesh,
        out_type=jax.ShapeDtypeStruct((NC, N, H), jnp.float32),
        scratch_types=[
            pltpu.VMEM_SHARED((N, H), jnp.float32),
            pltpu.VMEM((EDGES_PER_TILE,), jnp.int32),
            pltpu.VMEM((CHUNK,), jnp.int32),
            pltpu.VMEM((CHUNK,), jnp.int32),
            pltpu.VMEM((CHUNK, H), jnp.float32),
            pltpu.VMEM((CHUNK, H), jnp.float32),
            pltpu.SemaphoreType.DMA,
            pltpu.SemaphoreType.DMA,
            pltpu.SemaphoreType.DMA,
            pltpu.SemaphoreType.DMA,
        ],
    )
    def seg_kernel(x_hbm, src_hbm, dst_hbm, z_hbm, out_hbm,
                   acc, src_blk, dstv0, dstv1,
                   rows0, rows1, sem0, sem1, semd0, semd1):
        c = lax.axis_index("c")
        s = lax.axis_index("s")
        wid = c * NS + s
        r0 = s * ROWS_PER_TILE
        base = wid * EDGES_PER_TILE
        pltpu.sync_copy(src_hbm.at[pl.ds(base, EDGES_PER_TILE)], src_blk)
        pltpu.async_copy(dst_hbm.at[pl.ds(base, CHUNK)], dstv0, semd0)
        pltpu.async_copy(
            x_hbm.at[src_blk.at[pl.ds(0, CHUNK)]], rows0, sem0)
        for z in range(ROWS_PER_TILE // ZROWS):
            pltpu.sync_copy(z_hbm, acc.at[pl.ds(r0 + z * ZROWS, ZROWS)])

        @pl.when(s == NS - 1)
        def _zero_tail():
            pltpu.sync_copy(z_hbm.at[pl.ds(0, TAIL_ROWS)],
                            acc.at[pl.ds(NS * ROWS_PER_TILE, TAIL_ROWS)])

        plsc.subcore_barrier()

        def drain_rows(rows, sem):
            pltpu.make_async_copy(
                x_hbm.at[src_blk.at[pl.ds(0, CHUNK)]], rows, sem).wait()

        def drain_dst(dstv, semd):
            pltpu.make_async_copy(
                dst_hbm.at[pl.ds(base, CHUNK)], dstv, semd).wait()

        def pair(t, carry):
            ob = (2 * t + 1) * CHUNK
            pltpu.async_copy(dst_hbm.at[pl.ds(base + ob, CHUNK)], dstv1, semd1)
            pltpu.async_copy(
                x_hbm.at[src_blk.at[pl.ds(ob, CHUNK)]], rows1, sem1)
            drain_rows(rows0, sem0)
            drain_dst(dstv0, semd0)
            pltpu.sync_copy(rows0, acc.at[dstv0], add=True)
            oa = (2 * t + 2) * CHUNK
            pltpu.async_copy(dst_hbm.at[pl.ds(base + oa, CHUNK)], dstv0, semd0)
            pltpu.async_copy(
                x_hbm.at[src_blk.at[pl.ds(oa, CHUNK)]], rows0, sem0)
            drain_rows(rows1, sem1)
            drain_dst(dstv1, semd1)
            pltpu.sync_copy(rows1, acc.at[dstv1], add=True)
            return carry

        lax.fori_loop(0, (CHUNKS_PER_TILE - 1) // 2, pair, 0)
        drain_rows(rows0, sem0)
        drain_dst(dstv0, semd0)
        pltpu.sync_copy(rows0, acc.at[dstv0], add=True)
        plsc.subcore_barrier()
        pltpu.sync_copy(
            acc.at[pl.ds(r0, ROWS_PER_TILE)],
            out_hbm.at[c, pl.ds(r0, ROWS_PER_TILE)])

        @pl.when(s == NS - 1)
        def _out_tail():
            pltpu.sync_copy(
                acc.at[pl.ds(NS * ROWS_PER_TILE, TAIL_ROWS)],
                out_hbm.at[c, pl.ds(NS * ROWS_PER_TILE, TAIL_ROWS)])

    return seg_kernel(x, src, dst, zeros)


def _dense1_body(acc0, acc1, x, batch, features,
                 w1rel, b1rel, w1root, n1w, n1b,
                 m1w0, m1b0, m1ln0w, m1ln0b, m1w1, m1b1,
                 h_out, g_out):
    agg = acc0[...] + acc1[...]
    t = (jnp.dot(agg, w1rel[...], preferred_element_type=jnp.float32)
         + b1rel[...]
         + jnp.dot(x[...], w1root[...], preferred_element_type=jnp.float32))
    m = jnp.mean(t)
    d = t - m
    v = jnp.mean(d * d)
    h = jnp.maximum(d / jnp.sqrt(v + EPS) * n1w[...] + n1b[...], 0.0)
    h_out[...] = h

    bat = batch[...]
    rows = lax.broadcasted_iota(jnp.int32, (B, N), 0)
    onehot = (rows == bat).astype(jnp.float32)
    psum = jnp.dot(onehot, h, preferred_element_type=jnp.float32)
    cnt = jnp.sum(onehot, axis=1, keepdims=True)
    pooled = psum / jnp.maximum(cnt, 1.0)

    z0 = jnp.concatenate([pooled, features[...]], axis=1)
    z1 = (jnp.dot(z0, m1w0[...], preferred_element_type=jnp.float32)
          + m1b0[...])
    mu = jnp.mean(z1, axis=1, keepdims=True)
    dv = z1 - mu
    var = jnp.mean(dv * dv, axis=1, keepdims=True)
    z1 = jnp.maximum(dv / jnp.sqrt(var + EPS) * m1ln0w[...] + m1ln0b[...], 0.0)
    g_out[...] = (jnp.dot(z1, m1w1[...], preferred_element_type=jnp.float32)
                  + m1b1[...])


def _dense1(acc0, acc1, x, batch, features, w1rel, b1rel, w1root, n1w, n1b,
            m1w0, m1b0, m1ln0w, m1ln0b, m1w1, m1b1):
    return pl.pallas_call(
        _dense1_body,
        out_shape=[
            jax.ShapeDtypeStruct((N, H), jnp.float32),
            jax.ShapeDtypeStruct((B, 2 * H), jnp.float32),
        ],
    )(acc0, acc1, x, batch, features, w1rel, b1rel, w1root, n1w, n1b,
      m1w0, m1b0, m1ln0w, m1ln0b, m1w1, m1b1)


def _dense2_body(acc0, acc1, h, g, batch, features,
                 w2rel, b2rel, w2root, n2w, n2b,
                 m2w0, m2b0, m2ln0w, m2ln0b,
                 m2w1, m2b1, m2ln1w, m2ln1b,
                 m2w2, m2b2, m2ln2w, m2ln2b,
                 m2w3, m2b3, out):
    agg = acc0[...] + acc1[...]
    t = (jnp.dot(agg, w2rel[...], preferred_element_type=jnp.float32)
         + b2rel[...]
         + jnp.dot(h[...], w2root[...], preferred_element_type=jnp.float32))
    m = jnp.mean(t)
    d = t - m
    v = jnp.mean(d * d)
    h2 = jnp.maximum(d / jnp.sqrt(v + EPS) * n2w[...] + n2b[...], 0.0)

    bat = batch[...]
    rows = lax.broadcasted_iota(jnp.int32, (B, N), 0)
    onehot = (rows == bat).astype(jnp.float32)
    psum = jnp.dot(onehot, h2, preferred_element_type=jnp.float32)
    cnt = jnp.sum(onehot, axis=1, keepdims=True)
    pooled = psum / jnp.maximum(cnt, 1.0)

    z = jnp.concatenate([pooled, g[...], features[...]], axis=1)

    def ln_relu(zz, w, b):
        mu = jnp.mean(zz, axis=1, keepdims=True)
        dv = zz - mu
        var = jnp.mean(dv * dv, axis=1, keepdims=True)
        return jnp.maximum(dv / jnp.sqrt(var + EPS) * w + b, 0.0)

    z = ln_relu(jnp.dot(z, m2w0[...], preferred_element_type=jnp.float32)
                + m2b0[...], m2ln0w[...], m2ln0b[...])
    z = ln_relu(jnp.dot(z, m2w1[...], preferred_element_type=jnp.float32)
                + m2b1[...], m2ln1w[...], m2ln1b[...])
    z = ln_relu(jnp.dot(z, m2w2[...], preferred_element_type=jnp.float32)
                + m2b2[...], m2ln2w[...], m2ln2b[...])
    out[...] = (jnp.dot(z, m2w3[...], preferred_element_type=jnp.float32)
                + m2b3[...])


def _dense2(acc0, acc1, h, g, batch, features, w2rel, b2rel, w2root, n2w, n2b,
            m2w0, m2b0, m2ln0w, m2ln0b, m2w1, m2b1, m2ln1w, m2ln1b,
            m2w2, m2b2, m2ln2w, m2ln2b, m2w3, m2b3):
    return pl.pallas_call(
        _dense2_body,
        out_shape=jax.ShapeDtypeStruct((B, OUT), jnp.float32),
    )(acc0, acc1, h, g, batch, features, w2rel, b2rel, w2root, n2w, n2b,
      m2w0, m2b0, m2ln0w, m2ln0b, m2w1, m2b1, m2ln1w, m2ln1b,
      m2w2, m2b2, m2ln2w, m2ln2b, m2w3, m2b3)


def kernel(x, edge_index, features, batch,
           W1_rel, b1_rel, W1_root, n1_w, n1_b,
           m1_W0, m1_b0, m1_ln0_w, m1_ln0_b, m1_W1, m1_b1,
           W2_rel, b2_rel, W2_root, n2_w, n2_b,
           m2_W0, m2_b0, m2_ln0_w, m2_ln0_b, m2_W1, m2_b1,
           m2_ln1_w, m2_ln1_b, m2_W2, m2_b2, m2_ln2_w, m2_ln2_b,
           m2_W3, m2_b3):
    src = edge_index[0]
    dst = edge_index[1]
    zeros = jnp.zeros((ZROWS, H), jnp.float32)
    batch2d = batch.reshape(1, N)

    r1 = lambda a: a.reshape(1, -1)

    agg1 = _segment_sum_sc(x, src, dst, zeros)
    h, g = _dense1(agg1[0], agg1[1], x, batch2d, features,
                   W1_rel, r1(b1_rel), W1_root, r1(n1_w), r1(n1_b),
                   m1_W0, r1(m1_b0), r1(m1_ln0_w), r1(m1_ln0_b),
                   m1_W1, r1(m1_b1))
    agg2 = _segment_sum_sc(h, src, dst, zeros)
    out = _dense2(agg2[0], agg2[1], h, g, batch2d, features,
                  W2_rel, r1(b2_rel), W2_root, r1(n2_w), r1(n2_b),
                  m2_W0, r1(m2_b0), r1(m2_ln0_w), r1(m2_ln0_b),
                  m2_W1, r1(m2_b1), r1(m2_ln1_w), r1(m2_ln1_b),
                  m2_W2, r1(m2_b2), r1(m2_ln2_w), r1(m2_ln2_b),
                  m2_W3, r1(m2_b3))
    return out


# speedup vs baseline: 10.7559x; 1.0423x over previous
import functools

import jax
import jax.numpy as jnp
from jax import lax
from jax.experimental import pallas as pl
from jax.experimental.pallas import tpu as pltpu
from jax.experimental.pallas import tpu_sc as plsc

N = 10000
E = 320000
D_IN = 128
H = 128
F_G = 16
B = 64
OUT = 2
EPS = 1e-5

NC = 2
NS = 16
CHUNK = 80
EDGES_PER_TILE = E // (NC * NS)
CHUNKS_PER_TILE = EDGES_PER_TILE // CHUNK
ROWS_PER_TILE = 624
TAIL_ROWS = N - NS * ROWS_PER_TILE
ZROWS = 208


def _segment_sum_sc(x, edges_flat, zeros):
    mesh = plsc.VectorSubcoreMesh(core_axis_name="c", subcore_axis_name="s")

    @functools.partial(
        pl.kernel,
        mesh=mesh,
        out_type=jax.ShapeDtypeStruct((NC, N, H), jnp.float32),
        scratch_types=[
            pltpu.VMEM_SHARED((N, H), jnp.float32),
            pltpu.VMEM((EDGES_PER_TILE,), jnp.int32),
            pltpu.VMEM((CHUNK,), jnp.int32),
            pltpu.VMEM((CHUNK,), jnp.int32),
            pltpu.VMEM((CHUNK, H), jnp.float32),
            pltpu.VMEM((CHUNK, H), jnp.float32),
            pltpu.SemaphoreType.DMA,
            pltpu.SemaphoreType.DMA,
            pltpu.SemaphoreType.DMA,
            pltpu.SemaphoreType.DMA,
        ],
    )
    def seg_kernel(x_hbm, e_hbm, z_hbm, out_hbm,
                   acc, src_blk, dstv0, dstv1,
                   rows0, rows1, sem0, sem1, semd0, semd1):
        c = lax.axis_index("c")
        s = lax.axis_index("s")
        wid = c * NS + s
        r0 = s * ROWS_PER_TILE
        base = wid * EDGES_PER_TILE
        dbase = E + base
        pltpu.sync_copy(e_hbm.at[pl.ds(base, EDGES_PER_TILE)], src_blk)
        pltpu.async_copy(e_hbm.at[pl.ds(dbase, CHUNK)], dstv0, semd0)
        pltpu.async_copy(
            x_hbm.at[src_blk.at[pl.ds(0, CHUNK)]], rows0, sem0)
        for z in range(ROWS_PER_TILE // ZROWS):
            pltpu.sync_copy(z_hbm, acc.at[pl.ds(r0 + z * ZROWS, ZROWS)])

        @pl.when(s == NS - 1)
        def _zero_tail():
            pltpu.sync_copy(z_hbm.at[pl.ds(0, TAIL_ROWS)],
                            acc.at[pl.ds(NS * ROWS_PER_TILE, TAIL_ROWS)])

        plsc.subcore_barrier()

        def drain_rows(rows, sem):
            pltpu.make_async_copy(
                x_hbm.at[src_blk.at[pl.ds(0, CHUNK)]], rows, sem).wait()

        def drain_dst(dstv, semd):
            pltpu.make_async_copy(
                e_hbm.at[pl.ds(dbase, CHUNK)], dstv, semd).wait()

        def pair(t, carry):
            ob = (2 * t + 1) * CHUNK
            pltpu.async_copy(e_hbm.at[pl.ds(dbase + ob, CHUNK)], dstv1, semd1)
            pltpu.async_copy(
                x_hbm.at[src_blk.at[pl.ds(ob, CHUNK)]], rows1, sem1)
            drain_rows(rows0, sem0)
            drain_dst(dstv0, semd0)
            pltpu.sync_copy(rows0, acc.at[dstv0], add=True)
            oa = (2 * t + 2) * CHUNK
            pltpu.async_copy(e_hbm.at[pl.ds(dbase + oa, CHUNK)], dstv0, semd0)
            pltpu.async_copy(
                x_hbm.at[src_blk.at[pl.ds(oa, CHUNK)]], rows0, sem0)
            drain_rows(rows1, sem1)
            drain_dst(dstv1, semd1)
            pltpu.sync_copy(rows1, acc.at[dstv1], add=True)
            return carry

        lax.fori_loop(0, (CHUNKS_PER_TILE - 1) // 2, pair, 0)
        drain_rows(rows0, sem0)
        drain_dst(dstv0, semd0)
        pltpu.sync_copy(rows0, acc.at[dstv0], add=True)
        plsc.subcore_barrier()
        pltpu.sync_copy(
            acc.at[pl.ds(r0, ROWS_PER_TILE)],
            out_hbm.at[c, pl.ds(r0, ROWS_PER_TILE)])

        @pl.when(s == NS - 1)
        def _out_tail():
            pltpu.sync_copy(
                acc.at[pl.ds(NS * ROWS_PER_TILE, TAIL_ROWS)],
                out_hbm.at[c, pl.ds(NS * ROWS_PER_TILE, TAIL_ROWS)])

    return seg_kernel(x, edges_flat, zeros)


def _dense1_body(acc0, acc1, x, batch, features,
                 w1rel, b1rel, w1root, n1w, n1b,
                 m1w0, m1b0, m1ln0w, m1ln0b, m1w1, m1b1,
                 h_out, g_out):
    agg = acc0[...] + acc1[...]
    t = (jnp.dot(agg, w1rel[...], preferred_element_type=jnp.float32)
         + b1rel[...]
         + jnp.dot(x[...], w1root[...], preferred_element_type=jnp.float32))
    m = jnp.mean(t)
    d = t - m
    v = jnp.mean(d * d)
    h = jnp.maximum(d / jnp.sqrt(v + EPS) * n1w[...] + n1b[...], 0.0)
    h_out[...] = h

    bat = batch[...]
    rows = lax.broadcasted_iota(jnp.int32, (B, N), 0)
    onehot = (rows == bat).astype(jnp.float32)
    psum = jnp.dot(onehot, h, preferred_element_type=jnp.float32)
    cnt = jnp.sum(onehot, axis=1, keepdims=True)
    pooled = psum / jnp.maximum(cnt, 1.0)

    z0 = jnp.concatenate([pooled, features[...]], axis=1)
    z1 = (jnp.dot(z0, m1w0[...], preferred_element_type=jnp.float32)
          + m1b0[...])
    mu = jnp.mean(z1, axis=1, keepdims=True)
    dv = z1 - mu
    var = jnp.mean(dv * dv, axis=1, keepdims=True)
    z1 = jnp.maximum(dv / jnp.sqrt(var + EPS) * m1ln0w[...] + m1ln0b[...], 0.0)
    g_out[...] = (jnp.dot(z1, m1w1[...], preferred_element_type=jnp.float32)
                  + m1b1[...])


def _dense1(acc0, acc1, x, batch, features, w1rel, b1rel, w1root, n1w, n1b,
            m1w0, m1b0, m1ln0w, m1ln0b, m1w1, m1b1):
    return pl.pallas_call(
        _dense1_body,
        out_shape=[
            jax.ShapeDtypeStruct((N, H), jnp.float32),
            jax.ShapeDtypeStruct((B, 2 * H), jnp.float32),
        ],
    )(acc0, acc1, x, batch, features, w1rel, b1rel, w1root, n1w, n1b,
      m1w0, m1b0, m1ln0w, m1ln0b, m1w1, m1b1)


def _dense2_body(acc0, acc1, h, g, batch, features,
                 w2rel, b2rel, w2root, n2w, n2b,
                 m2w0, m2b0, m2ln0w, m2ln0b,
                 m2w1, m2b1, m2ln1w, m2ln1b,
                 m2w2, m2b2, m2ln2w, m2ln2b,
                 m2w3, m2b3, out):
    agg = acc0[...] + acc1[...]
    t = (jnp.dot(agg, w2rel[...], preferred_element_type=jnp.float32)
         + b2rel[...]
         + jnp.dot(h[...], w2root[...], preferred_element_type=jnp.float32))
    m = jnp.mean(t)
    d = t - m
    v = jnp.mean(d * d)
    h2 = jnp.maximum(d / jnp.sqrt(v + EPS) * n2w[...] + n2b[...], 0.0)

    bat = batch[...]
    rows = lax.broadcasted_iota(jnp.int32, (B, N), 0)
    onehot = (rows == bat).astype(jnp.float32)
    psum = jnp.dot(onehot, h2, preferred_element_type=jnp.float32)
    cnt = jnp.sum(onehot, axis=1, keepdims=True)
    pooled = psum / jnp.maximum(cnt, 1.0)

    z = jnp.concatenate([pooled, g[...], features[...]], axis=1)

    def ln_relu(zz, w, b):
        mu = jnp.mean(zz, axis=1, keepdims=True)
        dv = zz - mu
        var = jnp.mean(dv * dv, axis=1, keepdims=True)
        return jnp.maximum(dv / jnp.sqrt(var + EPS) * w + b, 0.0)

    z = ln_relu(jnp.dot(z, m2w0[...], preferred_element_type=jnp.float32)
                + m2b0[...], m2ln0w[...], m2ln0b[...])
    z = ln_relu(jnp.dot(z, m2w1[...], preferred_element_type=jnp.float32)
                + m2b1[...], m2ln1w[...], m2ln1b[...])
    z = ln_relu(jnp.dot(z, m2w2[...], preferred_element_type=jnp.float32)
                + m2b2[...], m2ln2w[...], m2ln2b[...])
    out[...] = (jnp.dot(z, m2w3[...], preferred_element_type=jnp.float32)
                + m2b3[...])


def _dense2(acc0, acc1, h, g, batch, features, w2rel, b2rel, w2root, n2w, n2b,
            m2w0, m2b0, m2ln0w, m2ln0b, m2w1, m2b1, m2ln1w, m2ln1b,
            m2w2, m2b2, m2ln2w, m2ln2b, m2w3, m2b3):
    return pl.pallas_call(
        _dense2_body,
        out_shape=jax.ShapeDtypeStruct((B, OUT), jnp.float32),
    )(acc0, acc1, h, g, batch, features, w2rel, b2rel, w2root, n2w, n2b,
      m2w0, m2b0, m2ln0w, m2ln0b, m2w1, m2b1, m2ln1w, m2ln1b,
      m2w2, m2b2, m2ln2w, m2ln2b, m2w3, m2b3)


def kernel(x, edge_index, features, batch,
           W1_rel, b1_rel, W1_root, n1_w, n1_b,
           m1_W0, m1_b0, m1_ln0_w, m1_ln0_b, m1_W1, m1_b1,
           W2_rel, b2_rel, W2_root, n2_w, n2_b,
           m2_W0, m2_b0, m2_ln0_w, m2_ln0_b, m2_W1, m2_b1,
           m2_ln1_w, m2_ln1_b, m2_W2, m2_b2, m2_ln2_w, m2_ln2_b,
           m2_W3, m2_b3):
    edges_flat = edge_index.reshape(-1)
    zeros = jnp.zeros((ZROWS, H), jnp.float32)
    batch2d = batch.reshape(1, N)

    r1 = lambda a: a.reshape(1, -1)

    agg1 = _segment_sum_sc(x, edges_flat, zeros)
    h, g = _dense1(agg1[0], agg1[1], x, batch2d, features,
                   W1_rel, r1(b1_rel), W1_root, r1(n1_w), r1(n1_b),
                   m1_W0, r1(m1_b0), r1(m1_ln0_w), r1(m1_ln0_b),
                   m1_W1, r1(m1_b1))
    agg2 = _segment_sum_sc(h, edges_flat, zeros)
    out = _dense2(agg2[0], agg2[1], h, g, batch2d, features,
                  W2_rel, r1(b2_rel), W2_root, r1(n2_w), r1(n2_b),
                  m2_W0, r1(m2_b0), r1(m2_ln0_w), r1(m2_ln0_b),
                  m2_W1, r1(m2_b1), r1(m2_ln1_w), r1(m2_ln1_b),
                  m2_W2, r1(m2_b2), r1(m2_ln2_w), r1(m2_ln2_b),
                  m2_W3, r1(m2_b3))
    return out


# speedup vs baseline: 12.2542x; 1.1393x over previous
import functools

import jax
import jax.numpy as jnp
from jax import lax
from jax.experimental import pallas as pl
from jax.experimental.pallas import tpu as pltpu
from jax.experimental.pallas import tpu_sc as plsc

N = 10000
E = 320000
D_IN = 128
H = 128
F_G = 16
B = 64
OUT = 2
EPS = 1e-5

NC = 2
NS = 16
NW = NC * NS
CHUNK = 80
EDGES_PER_TILE = E // NW
CHUNKS_PER_TILE = EDGES_PER_TILE // CHUNK
NB = 3
NITER = (CHUNKS_PER_TILE - 1) // NB
REM = CHUNKS_PER_TILE - NITER * NB
ROWS_PER_TILE = 624
TAIL_ROWS = N - NS * ROWS_PER_TILE
ZROWS = 208


def _segment_sum_sc(x, edges_flat, zeros):
    mesh = plsc.VectorSubcoreMesh(core_axis_name="c", subcore_axis_name="s")

    @functools.partial(
        pl.kernel,
        mesh=mesh,
        out_type=jax.ShapeDtypeStruct((NC, N, H), jnp.float32),
        scratch_types=[
            pltpu.VMEM_SHARED((N, H), jnp.float32),
            pltpu.VMEM((EDGES_PER_TILE,), jnp.int32),
        ]
        + [pltpu.VMEM((CHUNK,), jnp.int32)] * NB
        + [pltpu.VMEM((CHUNK, H), jnp.float32)] * NB
        + [pltpu.SemaphoreType.DMA] * (2 * NB + 1),
    )
    def seg_kernel(x_hbm, e_hbm, z_hbm, out_hbm, acc, src_blk, *ring):
        dstv = list(ring[:NB])
        rows = list(ring[NB:2 * NB])
        gsem = list(ring[2 * NB:3 * NB])
        ssem = list(ring[3 * NB:4 * NB])
        zsem = ring[4 * NB]
        c = lax.axis_index("c")
        s = lax.axis_index("s")
        wid = c * NS + s
        r0 = s * ROWS_PER_TILE
        base = wid * EDGES_PER_TILE
        dbase = E + base
        pltpu.sync_copy(e_hbm.at[pl.ds(base, EDGES_PER_TILE)], src_blk)

        def fire(j, b):
            pltpu.async_copy(
                e_hbm.at[pl.ds(dbase + j * CHUNK, CHUNK)], dstv[b], gsem[b])
            pltpu.async_copy(
                x_hbm.at[src_blk.at[pl.ds(j * CHUNK, CHUNK)]], rows[b],
                gsem[b])

        for b in range(NB):
            fire(b, b)
        for z in range(ROWS_PER_TILE // ZROWS):
            pltpu.async_copy(z_hbm, acc.at[pl.ds(r0 + z * ZROWS, ZROWS)],
                             zsem)

        @pl.when(s == NS - 1)
        def _zero_tail():
            pltpu.async_copy(z_hbm.at[pl.ds(0, TAIL_ROWS)],
                             acc.at[pl.ds(NS * ROWS_PER_TILE, TAIL_ROWS)],
                             zsem)

        for z in range(ROWS_PER_TILE // ZROWS):
            pltpu.make_async_copy(
                z_hbm, acc.at[pl.ds(r0 + z * ZROWS, ZROWS)], zsem).wait()

        @pl.when(s == NS - 1)
        def _zero_tail_wait():
            pltpu.make_async_copy(
                z_hbm.at[pl.ds(0, TAIL_ROWS)],
                acc.at[pl.ds(NS * ROWS_PER_TILE, TAIL_ROWS)], zsem).wait()

        plsc.subcore_barrier()

        def wait_fire(b):
            pltpu.make_async_copy(
                e_hbm.at[pl.ds(dbase, CHUNK)], dstv[b], gsem[b]).wait()
            pltpu.make_async_copy(
                x_hbm.at[src_blk.at[pl.ds(0, CHUNK)]], rows[b], gsem[b]).wait()

        def wait_scatter(b):
            pltpu.make_async_copy(rows[b], acc.at[dstv[b]], ssem[b]).wait()

        def group(t, carry):
            for b in range(NB):
                c = t * NB + b
                p = (b - 1) % NB
                wait_fire(b)
                if b == 0:
                    @pl.when(t > 0)
                    def _drain_refill():
                        wait_scatter(p)
                        fire(c - 1 + NB, p)
                else:
                    wait_scatter(p)
                    fire(c - 1 + NB, p)
                pltpu.async_copy(rows[b], acc.at[dstv[b]], ssem[b], add=True)
            return carry

        lax.fori_loop(0, NITER, group, 0)
        for r in range(REM):
            b = r % NB
            wait_fire(b)
            wait_scatter((b - 1) % NB)
            pltpu.async_copy(rows[b], acc.at[dstv[b]], ssem[b], add=True)
        wait_scatter((REM - 1) % NB)
        plsc.subcore_barrier()
        pltpu.sync_copy(
            acc.at[pl.ds(r0, ROWS_PER_TILE)],
            out_hbm.at[c, pl.ds(r0, ROWS_PER_TILE)])

        @pl.when(s == NS - 1)
        def _out_tail():
            pltpu.sync_copy(
                acc.at[pl.ds(NS * ROWS_PER_TILE, TAIL_ROWS)],
                out_hbm.at[c, pl.ds(NS * ROWS_PER_TILE, TAIL_ROWS)])

    return seg_kernel(x, edges_flat, zeros)


def _dense1_body(acc0, acc1, x, batch, features,
                 w1rel, b1rel, w1root, n1w, n1b,
                 m1w0, m1b0, m1ln0w, m1ln0b, m1w1, m1b1,
                 h_out, g_out):
    agg = acc0[...] + acc1[...]
    t = (jnp.dot(agg, w1rel[...], preferred_element_type=jnp.float32)
         + b1rel[...]
         + jnp.dot(x[...], w1root[...], preferred_element_type=jnp.float32))
    m = jnp.mean(t)
    d = t - m
    v = jnp.mean(d * d)
    h = jnp.maximum(d / jnp.sqrt(v + EPS) * n1w[...] + n1b[...], 0.0)
    h_out[...] = h

    bat = batch[...]
    rows = lax.broadcasted_iota(jnp.int32, (B, N), 0)
    onehot = (rows == bat).astype(jnp.float32)
    psum = jnp.dot(onehot, h, preferred_element_type=jnp.float32)
    cnt = jnp.sum(onehot, axis=1, keepdims=True)
    pooled = psum / jnp.maximum(cnt, 1.0)

    z0 = jnp.concatenate([pooled, features[...]], axis=1)
    z1 = (jnp.dot(z0, m1w0[...], preferred_element_type=jnp.float32)
          + m1b0[...])
    mu = jnp.mean(z1, axis=1, keepdims=True)
    dv = z1 - mu
    var = jnp.mean(dv * dv, axis=1, keepdims=True)
    z1 = jnp.maximum(dv / jnp.sqrt(var + EPS) * m1ln0w[...] + m1ln0b[...], 0.0)
    g_out[...] = (jnp.dot(z1, m1w1[...], preferred_element_type=jnp.float32)
                  + m1b1[...])


def _dense1(acc0, acc1, x, batch, features, w1rel, b1rel, w1root, n1w, n1b,
            m1w0, m1b0, m1ln0w, m1ln0b, m1w1, m1b1):
    return pl.pallas_call(
        _dense1_body,
        out_shape=[
            jax.ShapeDtypeStruct((N, H), jnp.float32),
            jax.ShapeDtypeStruct((B, 2 * H), jnp.float32),
        ],
    )(acc0, acc1, x, batch, features, w1rel, b1rel, w1root, n1w, n1b,
      m1w0, m1b0, m1ln0w, m1ln0b, m1w1, m1b1)


def _dense2_body(acc0, acc1, h, g, batch, features,
                 w2rel, b2rel, w2root, n2w, n2b,
                 m2w0, m2b0, m2ln0w, m2ln0b,
                 m2w1, m2b1, m2ln1w, m2ln1b,
                 m2w2, m2b2, m2ln2w, m2ln2b,
                 m2w3, m2b3, out):
    agg = acc0[...] + acc1[...]
    t = (jnp.dot(agg, w2rel[...], preferred_element_type=jnp.float32)
         + b2rel[...]
         + jnp.dot(h[...], w2root[...], preferred_element_type=jnp.float32))
    m = jnp.mean(t)
    d = t - m
    v = jnp.mean(d * d)
    h2 = jnp.maximum(d / jnp.sqrt(v + EPS) * n2w[...] + n2b[...], 0.0)

    bat = batch[...]
    rows = lax.broadcasted_iota(jnp.int32, (B, N), 0)
    onehot = (rows == bat).astype(jnp.float32)
    psum = jnp.dot(onehot, h2, preferred_element_type=jnp.float32)
    cnt = jnp.sum(onehot, axis=1, keepdims=True)
    pooled = psum / jnp.maximum(cnt, 1.0)

    z = jnp.concatenate([pooled, g[...], features[...]], axis=1)

    def ln_relu(zz, w, b):
        mu = jnp.mean(zz, axis=1, keepdims=True)
        dv = zz - mu
        var = jnp.mean(dv * dv, axis=1, keepdims=True)
        return jnp.maximum(dv / jnp.sqrt(var + EPS) * w + b, 0.0)

    z = ln_relu(jnp.dot(z, m2w0[...], preferred_element_type=jnp.float32)
                + m2b0[...], m2ln0w[...], m2ln0b[...])
    z = ln_relu(jnp.dot(z, m2w1[...], preferred_element_type=jnp.float32)
                + m2b1[...], m2ln1w[...], m2ln1b[...])
    z = ln_relu(jnp.dot(z, m2w2[...], preferred_element_type=jnp.float32)
                + m2b2[...], m2ln2w[...], m2ln2b[...])
    out[...] = (jnp.dot(z, m2w3[...], preferred_element_type=jnp.float32)
                + m2b3[...])


def _dense2(acc0, acc1, h, g, batch, features, w2rel, b2rel, w2root, n2w, n2b,
            m2w0, m2b0, m2ln0w, m2ln0b, m2w1, m2b1, m2ln1w, m2ln1b,
            m2w2, m2b2, m2ln2w, m2ln2b, m2w3, m2b3):
    return pl.pallas_call(
        _dense2_body,
        out_shape=jax.ShapeDtypeStruct((B, OUT), jnp.float32),
    )(acc0, acc1, h, g, batch, features, w2rel, b2rel, w2root, n2w, n2b,
      m2w0, m2b0, m2ln0w, m2ln0b, m2w1, m2b1, m2ln1w, m2ln1b,
      m2w2, m2b2, m2ln2w, m2ln2b, m2w3, m2b3)


def kernel(x, edge_index, features, batch,
           W1_rel, b1_rel, W1_root, n1_w, n1_b,
           m1_W0, m1_b0, m1_ln0_w, m1_ln0_b, m1_W1, m1_b1,
           W2_rel, b2_rel, W2_root, n2_w, n2_b,
           m2_W0, m2_b0, m2_ln0_w, m2_ln0_b, m2_W1, m2_b1,
           m2_ln1_w, m2_ln1_b, m2_W2, m2_b2, m2_ln2_w, m2_ln2_b,
           m2_W3, m2_b3):
    edges_flat = edge_index.reshape(-1)
    zeros = jnp.zeros((ZROWS, H), jnp.float32)
    batch2d = batch.reshape(1, N)

    r1 = lambda a: a.reshape(1, -1)

    agg1 = _segment_sum_sc(x, edges_flat, zeros)
    h, g = _dense1(agg1[0], agg1[1], x, batch2d, features,
                   W1_rel, r1(b1_rel), W1_root, r1(n1_w), r1(n1_b),
                   m1_W0, r1(m1_b0), r1(m1_ln0_w), r1(m1_ln0_b),
                   m1_W1, r1(m1_b1))
    agg2 = _segment_sum_sc(h, edges_flat, zeros)
    out = _dense2(agg2[0], agg2[1], h, g, batch2d, features,
                  W2_rel, r1(b2_rel), W2_root, r1(n2_w), r1(n2_b),
                  m2_W0, r1(m2_b0), r1(m2_ln0_w), r1(m2_ln0_b),
                  m2_W1, r1(m2_b1), r1(m2_ln1_w), r1(m2_ln1_b),
                  m2_W2, r1(m2_b2), r1(m2_ln2_w), r1(m2_ln2_b),
                  m2_W3, r1(m2_b3))
    return out
